# baseline (speedup 1.0000x reference)
import functools

import jax
import jax.numpy as jnp
from jax import lax
from jax.experimental import pallas as pl
from jax.experimental.pallas import tpu as pltpu
from jax.experimental.pallas import tpu_sc as plsc

K = 64
_L = 16
_NW = 32
_NC = 2


def _encode_body(x_ref, w_ref, pb_ref, lb_ref, out_ref):
    xb = x_ref[...] - pb_ref[...]
    acts = lax.dot_general(xb, w_ref[...], (((1,), (1,)), ((), ())),
                           preferred_element_type=jnp.float32)
    out_ref[...] = acts + lb_ref[...]


def _encode(x, W_enc, pre_bias, latent_bias, *, br=1024, bh=2048):
    n, d = x.shape
    h = W_enc.shape[0]
    grid = (n // br, h // bh)
    return pl.pallas_call(
        _encode_body,
        grid=grid,
        in_specs=[
            pl.BlockSpec((br, d), lambda r, hh: (r, 0)),
            pl.BlockSpec((bh, d), lambda r, hh: (hh, 0)),
            pl.BlockSpec((1, d), lambda r, hh: (0, 0)),
            pl.BlockSpec((1, bh), lambda r, hh: (0, hh)),
        ],
        out_specs=pl.BlockSpec((br, bh), lambda r, hh: (r, hh)),
        out_shape=jax.ShapeDtypeStruct((n, h), jnp.float32),
    )(x, W_enc, pre_bias.reshape(1, d), latent_bias.reshape(1, h))


def _iota():
    return lax.iota(jnp.int32, _L)


def _cmp_lt(av, ai, bv, bi):
    return (av < bv) | ((av == bv) & (ai > bi))


def _permute(x, idx):
    dnums = lax.GatherDimensionNumbers(
        offset_dims=(), collapsed_slice_dims=(0,), start_index_map=(0,))
    return lax.gather(x, idx[:, None], dnums, slice_sizes=(1,),
                      unique_indices=True,
                      mode=lax.GatherScatterMode.PROMISE_IN_BOUNDS)


def _tree_sum(x):
    for j in (1, 2, 4, 8):
        x = x + _permute(x, _iota() ^ j)
    return x


def _tree_min(x):
    for j in (1, 2, 4, 8):
        x = jnp.minimum(x, _permute(x, _iota() ^ j))
    return x


def _sort16(v, i):
    for k in (2, 4, 8, 16):
        j = k // 2
        while j >= 1:
            desc_mask = (_iota() & k) == 0
            v, i = _cx_intra(v, i, j, desc_mask)
            j //= 2
    return v, i


def _cx_intra(v, i, j, desc_mask):
    perm = _iota() ^ j
    pv, pi = _permute(v, perm), _permute(i, perm)
    rl = _cmp_lt(v, i, pv, pi)
    is_lo = (_iota() & j) == 0
    take = rl ^ is_lo ^ desc_mask
    return jnp.where(take, pv, v), jnp.where(take, pi, i)


def _cx_inter(Sv, Si, lo, hi, desc):
    if desc:
        swap = _cmp_lt(Sv[lo], Si[lo], Sv[hi], Si[hi])
    else:
        swap = _cmp_lt(Sv[hi], Si[hi], Sv[lo], Si[lo])
    nlo_v = jnp.where(swap, Sv[hi], Sv[lo])
    nlo_i = jnp.where(swap, Si[hi], Si[lo])
    nhi_v = jnp.where(swap, Sv[lo], Sv[hi])
    nhi_i = jnp.where(swap, Si[lo], Si[hi])
    Sv[lo], Si[lo], Sv[hi], Si[hi] = nlo_v, nlo_i, nhi_v, nhi_i


def _sort64(Sv, Si):
    Sv, Si = list(Sv), list(Si)
    for k in (2, 4, 8, 16, 32, 64):
        j = k // 2
        while j >= 1:
            if j >= _L:
                g = j // _L
                for lo in range(4):
                    if lo & g:
                        continue
                    _cx_inter(Sv, Si, lo, lo | g, ((lo * _L) & k) == 0)
            else:
                for g in range(4):
                    desc_mask = ((_iota() + g * _L) & k) == 0
                    Sv[g], Si[g] = _cx_intra(Sv[g], Si[g], j, desc_mask)
            j //= 2
    return Sv, Si


def _merge64(Sv, Si, Bv, Bi):
    rev = _L - 1 - _iota()
    Wv, Wi = [None] * 4, [None] * 4
    for g in range(4):
        pv = _permute(Bv[3 - g], rev)
        pi = _permute(Bi[3 - g], rev)
        rl = _cmp_lt(Sv[g], Si[g], pv, pi)
        Wv[g] = jnp.where(rl, pv, Sv[g])
        Wi[g] = jnp.where(rl, pi, Si[g])
    for j in (32, 16):
        g = j // _L
        for lo in range(4):
            if lo & g:
                continue
            _cx_inter(Wv, Wi, lo, lo | g, True)
    ones = _iota() >= 0
    for j in (8, 4, 2, 1):
        for g in range(4):
            Wv[g], Wi[g] = _cx_intra(Wv[g], Wi[g], j, ones)
    return Wv, Wi


def _flatten_state(Sv, Si, T, pos):
    return (*Sv, *Si, T, pos)


def _unflatten_state(c):
    return list(c[0:4]), list(c[4:8]), c[8], c[9]


def _make_sc_topk(n, h):
    rows_per_w = n // _NW
    nchunk = h // _L
    mesh = plsc.VectorSubcoreMesh(core_axis_name="c", subcore_axis_name="s")

    @functools.partial(
        pl.kernel,
        out_type=[jax.ShapeDtypeStruct((n, K), jnp.float32),
                  jax.ShapeDtypeStruct((n, K), jnp.int32)],
        mesh=mesh,
        scratch_types=[
            pltpu.VMEM((2, h), jnp.float32),
            pltpu.VMEM((K + _L,), jnp.float32),
            pltpu.VMEM((K + _L,), jnp.int32),
            pltpu.VMEM((K,), jnp.float32),
            pltpu.VMEM((K,), jnp.int32),
            pltpu.SemaphoreType.DMA,
            pltpu.SemaphoreType.DMA,
        ],
    )
    def sc_topk(pre_hbm, tv_hbm, ti_hbm, rowbuf, bv, bi, ov, oi, sem0, sem1):
        wid = lax.axis_index("s") * _NC + lax.axis_index("c")
        base = wid * rows_per_w
        sems = (sem0, sem1)

        pltpu.async_copy(pre_hbm.at[base], rowbuf.at[0], sem0)

        def do_merge(pos):
            Sv = [ov[pl.ds(g * _L, _L)] for g in range(4)]
            Si = [oi[pl.ds(g * _L, _L)] for g in range(4)]
            Bv, Bi = [], []
            for g in range(4):
                lane = _iota() + g * _L
                keep = lane < pos
                Bv.append(jnp.where(keep, bv[pl.ds(g * _L, _L)], -1.0))
                Bi.append(jnp.where(keep, bi[pl.ds(g * _L, _L)], lane))
            Bv, Bi = _sort64(Bv, Bi)
            Wv, Wi = _merge64(Sv, Si, Bv, Bi)
            for g in range(4):
                ov[pl.ds(g * _L, _L)] = Wv[g]
                oi[pl.ds(g * _L, _L)] = Wi[g]
            return Wv[3][_L - 1]

        def process_row(r, buf):
            @pl.when(r + 1 < rows_per_w)
            def _():
                pltpu.async_copy(pre_hbm.at[base + r + 1],
                                 rowbuf.at[1 - buf], sems[1 - buf])

            pltpu.make_async_copy(pre_hbm.at[base + r],
                                  rowbuf.at[buf], sems[buf]).wait()

            for g in range(4):
                ov[pl.ds(g * _L, _L)] = jnp.full((_L,), -1.0, jnp.float32)
                oi[pl.ds(g * _L, _L)] = _iota() + g * _L

            def chunk_body(c, carry):
                T, pos = carry
                v = jnp.maximum(rowbuf[buf, pl.ds(c * _L, _L)], 0.0)
                m = v > T
                cnt = _tree_sum(jnp.where(m, 1, 0))[0]

                def no_hit(op):
                    return op

                def hit(op):
                    T, pos = op
                    iv = _iota() + c * _L
                    vm = jnp.where(m, v, -1.0)

                    def single(p):
                        l = _tree_min(jnp.where(m, _iota(), _L))[0]
                        perm = (_iota() + l) & (_L - 1)
                        bv[pl.ds(p, _L)] = _permute(vm, perm)
                        bi[pl.ds(p, _L)] = _permute(iv, perm)
                        return p + 1

                    def multi(p):
                        sv, si = _sort16(vm, iv)
                        bv[pl.ds(p, _L)] = sv
                        bi[pl.ds(p, _L)] = si
                        return p + cnt

                    pos = lax.cond(cnt == 1, single, multi, pos)

                    def with_merge(op2):
                        _, pos = op2
                        return (do_merge(pos), jnp.int32(0))

                    def no_merge(op2):
                        return op2

                    return lax.cond(pos > (K - _L), with_merge, no_merge,
                                    (T, pos))

                return lax.cond(cnt > 0, hit, no_hit, (T, pos))

            init = (jnp.float32(-1.0), jnp.int32(0))
            _, pos = lax.fori_loop(0, nchunk, chunk_body, init)
            do_merge(pos)
            pltpu.sync_copy(ov, tv_hbm.at[base + r])
            pltpu.sync_copy(oi, ti_hbm.at[base + r])

        def outer(r2, _):
            process_row(r2 * 2, 0)
            process_row(r2 * 2 + 1, 1)
            return 0

        lax.fori_loop(0, rows_per_w // 2, outer, 0)

    return sc_topk


def _decode_body(bk, pre_ref, wd_ref, pb_ref, t_ref, i_ref, sc_ref, out_ref):
    kk = pl.program_id(1)
    a = jnp.maximum(pre_ref[...], 0.0)
    col = lax.broadcasted_iota(jnp.int32, a.shape, 1) + kk * bk
    t = t_ref[...]
    i64 = i_ref[...]
    mask = (a > t) | ((a == t) & (col <= i64))
    sc = jnp.where(mask, a, 0.0)
    sc_ref[...] = sc
    part = lax.dot_general(sc, wd_ref[...], (((1,), (1,)), ((), ())),
                           preferred_element_type=jnp.float32)

    @pl.when(kk == 0)
    def _():
        out_ref[...] = part + pb_ref[...]

    @pl.when(kk != 0)
    def _():
        out_ref[...] = out_ref[...] + part


def _decode(pre_acts, W_dec, pre_bias, t64, i64, *, br=512, bk=2048):
    n, h = pre_acts.shape
    d = W_dec.shape[0]
    grid = (n // br, h // bk)
    return pl.pallas_call(
        functools.partial(_decode_body, bk),
        grid=grid,
        in_specs=[
            pl.BlockSpec((br, bk), lambda r, kk: (r, kk)),
            pl.BlockSpec((d, bk), lambda r, kk: (0, kk)),
            pl.BlockSpec((1, d), lambda r, kk: (0, 0)),
            pl.BlockSpec((br, 1), lambda r, kk: (r, 0)),
            pl.BlockSpec((br, 1), lambda r, kk: (r, 0)),
        ],
        out_specs=[
            pl.BlockSpec((br, bk), lambda r, kk: (r, kk)),
            pl.BlockSpec((br, d), lambda r, kk: (r, 0)),
        ],
        out_shape=[
            jax.ShapeDtypeStruct((n, h), jnp.float32),
            jax.ShapeDtypeStruct((n, d), jnp.float32),
        ],
        compiler_params=pltpu.CompilerParams(
            dimension_semantics=("parallel", "arbitrary")),
    )(pre_acts, W_dec, pre_bias.reshape(1, d), t64, i64)


def kernel(x, W_enc, W_dec, pre_bias, latent_bias):
    n, d = x.shape
    h = W_enc.shape[0]
    pre_acts = _encode(x, W_enc, pre_bias, latent_bias)
    topk_values, topk_indices = _make_sc_topk(n, h)(pre_acts)
    t64 = topk_values[:, K - 1:K]
    i64 = topk_indices[:, K - 1:K]
    sparse_code, reconstruction = _decode(pre_acts, W_dec, pre_bias, t64, i64)
    return (reconstruction, sparse_code, pre_acts, topk_values, topk_indices)
